# Initial kernel scaffold; baseline (speedup 1.0000x reference)
import jax
import jax.numpy as jnp
from jax.experimental import pallas as pl


def kernel(x, p, edge_index, Wg1, bg1, Wg2, bg2, Wl1, bl1, Wl2, bl2):
    raise NotImplementedError("write your pallas kernel here")


# speedup vs baseline: 13.2468x; 13.2468x over previous
import functools

import jax
import jax.numpy as jnp
from jax import lax
from jax.experimental import pallas as pl
from jax.experimental.pallas import tpu as pltpu
from jax.experimental.pallas import tpu_sc as plsc

NC = 2
NS = 16
NW = NC * NS
L = 16


def _d1_body(x_ref, wg1_ref, bg1_ref, wg2_ref, bg2_ref, wl1a_ref,
             mg_ref, h1l_ref):
    xb = x_ref[...]
    h = jnp.maximum(
        jnp.dot(xb, wg1_ref[...], preferred_element_type=jnp.float32)
        + bg1_ref[...], 0.0)
    mg_ref[...] = (jnp.dot(h, wg2_ref[...], preferred_element_type=jnp.float32)
                   + bg2_ref[...])
    h1l_ref[...] = jnp.dot(xb, wl1a_ref[...],
                           preferred_element_type=jnp.float32)


def _d2a_body(cpart_ref, mg_ref, wl1b_ref, bl1_ref, out_ref):
    c = jnp.sum(cpart_ref[...], axis=0, keepdims=True)
    s = jnp.dot(c, mg_ref[...], preferred_element_type=jnp.float32)
    out_ref[...] = (jnp.dot(s, wl1b_ref[...],
                            preferred_element_type=jnp.float32)
                    + bl1_ref[...])


def _d2b_body(h1l_ref, bias_ref, wl2_ref, bl2_ref, out_ref):
    h = jnp.maximum(h1l_ref[...] + bias_ref[...], 0.0)
    out_ref[...] = (jnp.dot(h, wl2_ref[...],
                            preferred_element_type=jnp.float32)
                    + bl2_ref[...])


def _sum_body(a_ref, b_ref, o_ref):
    o_ref[...] = a_ref[0] + b_ref[0]


def _make_sc_c(n_nodes, e_per_w):
    mesh = plsc.VectorSubcoreMesh(core_axis_name="c", subcore_axis_name="s",
                                  num_cores=NC, num_subcores=NS)

    @functools.partial(
        pl.kernel,
        out_type=jax.ShapeDtypeStruct((NW, n_nodes), jnp.float32),
        mesh=mesh,
        scratch_types=[
            pltpu.VMEM((n_nodes,), jnp.float32),
            pltpu.VMEM((e_per_w,), jnp.int32),
            pltpu.VMEM((e_per_w,), jnp.int32),
            pltpu.VMEM((n_nodes,), jnp.float32),
        ],
        compiler_params=pltpu.CompilerParams(needs_layout_passes=False),
    )
    def sc_c(src_hbm, dst_hbm, evals_hbm, out_hbm, evals_v, src_v, dst_v, c_v):
        cid = lax.axis_index("c")
        sid = lax.axis_index("s")
        wid = sid * NC + cid
        pltpu.sync_copy(evals_hbm, evals_v)
        pltpu.sync_copy(src_hbm.at[wid], src_v)
        pltpu.sync_copy(dst_hbm.at[wid], dst_v)

        def zero_body(i, carry):
            c_v[pl.ds(i * L, L)] = jnp.zeros((L,), jnp.float32)
            return carry
        lax.fori_loop(0, n_nodes // L, zero_body, 0)

        def body(i, carry):
            sv = src_v[pl.ds(i * L, L)]
            dv = dst_v[pl.ds(i * L, L)]
            vals = plsc.load_gather(evals_v, [dv])
            plsc.addupdate_scatter(c_v, [sv], vals)
            return carry
        lax.fori_loop(0, e_per_w // L, body, 0)
        pltpu.sync_copy(c_v, out_hbm.at[wid])

    return sc_c


def _make_sc_w(n_nodes, d_out, n_chunks, k):
    mesh = plsc.VectorSubcoreMesh(core_axis_name="c", subcore_axis_name="s",
                                  num_cores=NC, num_subcores=NS)
    zk = 80
    n_zchunks = n_nodes // zk

    @functools.partial(
        pl.kernel,
        out_type=jax.ShapeDtypeStruct((NC, n_nodes, d_out), jnp.float32),
        mesh=mesh,
        scratch_types=[
            pltpu.VMEM((n_chunks, k), jnp.int32),
            pltpu.VMEM((n_chunks, k), jnp.int32),
            pltpu.VMEM((k, d_out), jnp.float32),
            pltpu.VMEM_SHARED((n_nodes, d_out), jnp.float32),
            pltpu.SemaphoreType.DMA,
        ],
        compiler_params=pltpu.CompilerParams(needs_layout_passes=False),
    )
    def sc_w(src_hbm, dst_hbm, ml_hbm, out_hbm, src_v, dst_v, rows_v,
             shw, sem):
        cid = lax.axis_index("c")
        sid = lax.axis_index("s")
        wid = sid * NC + cid
        pltpu.sync_copy(src_hbm.at[wid], src_v)
        pltpu.sync_copy(dst_hbm.at[wid], dst_v)

        def zero_body(i, carry):
            rows_v[i // (d_out // L), pl.ds((i % (d_out // L)) * L, L)] = (
                jnp.zeros((L,), jnp.float32))
            return carry
        lax.fori_loop(0, k * (d_out // L), zero_body, 0)
        n_rounds = (n_zchunks + NS - 1) // NS
        for t in range(n_rounds):
            cj = sid + t * NS

            @pl.when(cj < n_zchunks)
            def _():
                pltpu.sync_copy(rows_v.at[pl.ds(0, zk)],
                                shw.at[pl.ds(cj * zk, zk)])
        plsc.subcore_barrier()

        def chunk_body(j, carry):
            pltpu.async_copy(ml_hbm.at[src_v.at[j]], rows_v, sem).wait()
            pltpu.sync_copy(rows_v, shw.at[dst_v.at[j]], add=True)
            return carry
        lax.fori_loop(0, n_chunks, chunk_body, 0)
        plsc.subcore_barrier()

        for t in range(n_rounds):
            cj = sid + t * NS

            @pl.when(cj < n_zchunks)
            def _():
                pltpu.sync_copy(shw.at[pl.ds(cj * zk, zk)],
                                rows_v.at[pl.ds(0, zk)])
                pltpu.sync_copy(rows_v.at[pl.ds(0, zk)],
                                out_hbm.at[cid, pl.ds(cj * zk, zk)])

    return sc_w


def kernel(x, p, edge_index, Wg1, bg1, Wg2, bg2, Wl1, bl1, Wl2, bl2):
    n, d = x.shape
    e = edge_index.shape[1]
    dg_o = Wg2.shape[1]
    dl_h = Wl1.shape[1]
    dl_o = Wl2.shape[1]

    src = edge_index[0]
    dst = edge_index[1]
    evals = p[:, 0]
    Wl1a = Wl1[:d]
    Wl1b = Wl1[d:]

    e_per_w = e // NW
    k = 125
    n_chunks = e_per_w // k
    src_w = src.reshape(NW, e_per_w)
    dst_w = dst.reshape(NW, e_per_w)
    src_c = src.reshape(NW, n_chunks, k)
    dst_c = dst.reshape(NW, n_chunks, k)

    bn = 2000

    m_g, h1l = pl.pallas_call(
        _d1_body,
        grid=(n // bn,),
        in_specs=[
            pl.BlockSpec((bn, d), lambda i: (i, 0)),
            pl.BlockSpec(Wg1.shape, lambda i: (0, 0)),
            pl.BlockSpec((1, Wg1.shape[1]), lambda i: (0, 0)),
            pl.BlockSpec(Wg2.shape, lambda i: (0, 0)),
            pl.BlockSpec((1, dg_o), lambda i: (0, 0)),
            pl.BlockSpec((d, dl_h), lambda i: (0, 0)),
        ],
        out_specs=[
            pl.BlockSpec((bn, dg_o), lambda i: (i, 0)),
            pl.BlockSpec((bn, dl_h), lambda i: (i, 0)),
        ],
        out_shape=[
            jax.ShapeDtypeStruct((n, dg_o), jnp.float32),
            jax.ShapeDtypeStruct((n, dl_h), jnp.float32),
        ],
    )(x, Wg1, bg1.reshape(1, -1), Wg2, bg2.reshape(1, -1), Wl1a)

    c_part = _make_sc_c(n, e_per_w)(src_w, dst_w, evals)

    bias2 = pl.pallas_call(
        _d2a_body,
        in_specs=[
            pl.BlockSpec((NW, n), lambda: (0, 0)),
            pl.BlockSpec((n, dg_o), lambda: (0, 0)),
            pl.BlockSpec((dg_o, dl_h), lambda: (0, 0)),
            pl.BlockSpec((1, dl_h), lambda: (0, 0)),
        ],
        out_specs=pl.BlockSpec((1, dl_h), lambda: (0, 0)),
        out_shape=jax.ShapeDtypeStruct((1, dl_h), jnp.float32),
    )(c_part, m_g, Wl1b, bl1.reshape(1, -1))

    m_l = pl.pallas_call(
        _d2b_body,
        grid=(n // bn,),
        in_specs=[
            pl.BlockSpec((bn, dl_h), lambda i: (i, 0)),
            pl.BlockSpec((1, dl_h), lambda i: (0, 0)),
            pl.BlockSpec((dl_h, dl_o), lambda i: (0, 0)),
            pl.BlockSpec((1, dl_o), lambda i: (0, 0)),
        ],
        out_specs=pl.BlockSpec((bn, dl_o), lambda i: (i, 0)),
        out_shape=jax.ShapeDtypeStruct((n, dl_o), jnp.float32),
    )(h1l, bias2, Wl2, bl2.reshape(1, -1))

    w_part = _make_sc_w(n, dl_o, n_chunks, k)(src_c, dst_c, m_l)

    w = pl.pallas_call(
        _sum_body,
        grid=(n // bn,),
        in_specs=[
            pl.BlockSpec((1, bn, dl_o), lambda i: (0, i, 0)),
            pl.BlockSpec((1, bn, dl_o), lambda i: (1, i, 0)),
        ],
        out_specs=pl.BlockSpec((bn, dl_o), lambda i: (i, 0)),
        out_shape=jax.ShapeDtypeStruct((n, dl_o), jnp.float32),
    )(w_part, w_part)

    return w


# speedup vs baseline: 17.6985x; 1.3361x over previous
import functools

import jax
import jax.numpy as jnp
from jax import lax
from jax.experimental import pallas as pl
from jax.experimental.pallas import tpu as pltpu
from jax.experimental.pallas import tpu_sc as plsc

NC = 2
NS = 16
NW = NC * NS
L = 16


def _d1_body(x_ref, wg1_ref, bg1_ref, wg2_ref, bg2_ref, wl1a_ref,
             mg_ref, h1l_ref):
    xb = x_ref[...]
    h = jnp.maximum(
        jnp.dot(xb, wg1_ref[...], preferred_element_type=jnp.float32)
        + bg1_ref[...], 0.0)
    mg_ref[...] = (jnp.dot(h, wg2_ref[...], preferred_element_type=jnp.float32)
                   + bg2_ref[...])
    h1l_ref[...] = jnp.dot(xb, wl1a_ref[...],
                           preferred_element_type=jnp.float32)


def _d2a_body(cpart_ref, mg_ref, wl1b_ref, bl1_ref, out_ref):
    c = jnp.sum(cpart_ref[...], axis=0, keepdims=True)
    s = jnp.dot(c, mg_ref[...], preferred_element_type=jnp.float32)
    out_ref[...] = (jnp.dot(s, wl1b_ref[...],
                            preferred_element_type=jnp.float32)
                    + bl1_ref[...])


def _d2b_body(h1l_ref, bias_ref, wl2_ref, bl2_ref, out_ref):
    h = jnp.maximum(h1l_ref[...] + bias_ref[...], 0.0)
    out_ref[...] = (jnp.dot(h, wl2_ref[...],
                            preferred_element_type=jnp.float32)
                    + bl2_ref[...])


def _sum_body(a_ref, b_ref, o_ref):
    o_ref[...] = a_ref[0] + b_ref[0]


def _make_sc_c(n_nodes, e_per_w):
    mesh = plsc.VectorSubcoreMesh(core_axis_name="c", subcore_axis_name="s",
                                  num_cores=NC, num_subcores=NS)

    @functools.partial(
        pl.kernel,
        out_type=jax.ShapeDtypeStruct((NW, n_nodes), jnp.float32),
        mesh=mesh,
        scratch_types=[
            pltpu.VMEM((n_nodes,), jnp.float32),
            pltpu.VMEM((e_per_w,), jnp.int32),
            pltpu.VMEM((e_per_w,), jnp.int32),
            pltpu.VMEM((n_nodes,), jnp.float32),
        ],
        compiler_params=pltpu.CompilerParams(needs_layout_passes=False),
    )
    def sc_c(src_hbm, dst_hbm, evals_hbm, out_hbm, evals_v, src_v, dst_v, c_v):
        cid = lax.axis_index("c")
        sid = lax.axis_index("s")
        wid = sid * NC + cid
        pltpu.sync_copy(evals_hbm, evals_v)
        pltpu.sync_copy(src_hbm.at[wid], src_v)
        pltpu.sync_copy(dst_hbm.at[wid], dst_v)

        def zero_body(i, carry):
            c_v[pl.ds(i * L, L)] = jnp.zeros((L,), jnp.float32)
            return carry
        lax.fori_loop(0, n_nodes // L, zero_body, 0)

        def body(i, carry):
            sv = src_v[pl.ds(i * L, L)]
            dv = dst_v[pl.ds(i * L, L)]
            vals = plsc.load_gather(evals_v, [dv])
            plsc.addupdate_scatter(c_v, [sv], vals)
            return carry
        lax.fori_loop(0, e_per_w // L, body, 0)
        pltpu.sync_copy(c_v, out_hbm.at[wid])

    return sc_c


def _make_sc_w(n_nodes, d_out, n_chunks, k):
    mesh = plsc.VectorSubcoreMesh(core_axis_name="c", subcore_axis_name="s",
                                  num_cores=NC, num_subcores=NS)
    zk = 80
    n_zchunks = n_nodes // zk

    @functools.partial(
        pl.kernel,
        out_type=jax.ShapeDtypeStruct((NC, n_nodes, d_out), jnp.float32),
        mesh=mesh,
        scratch_types=[
            pltpu.VMEM((n_chunks * k,), jnp.int32),
            pltpu.VMEM((n_chunks, k), jnp.int32),
            pltpu.VMEM((k, d_out), jnp.float32),
            pltpu.VMEM((k, d_out), jnp.float32),
            pltpu.VMEM_SHARED((n_nodes, d_out), jnp.float32),
            pltpu.SemaphoreType.DMA,
            pltpu.SemaphoreType.DMA,
        ],
        compiler_params=pltpu.CompilerParams(needs_layout_passes=False),
    )
    def sc_w(src_hbm, dst_hbm, ml_hbm, out_hbm, src_v, dst_v, rows0, rows1,
             shw, sem0, sem1):
        cid = lax.axis_index("c")
        sid = lax.axis_index("s")
        wid = sid * NC + cid
        pltpu.sync_copy(src_hbm.at[wid], src_v)
        pltpu.sync_copy(dst_hbm.at[wid], dst_v)

        def zero_body(i, carry):
            rows0[i // (d_out // L), pl.ds((i % (d_out // L)) * L, L)] = (
                jnp.zeros((L,), jnp.float32))
            return carry
        lax.fori_loop(0, k * (d_out // L), zero_body, 0)
        n_rounds = (n_zchunks + NS - 1) // NS
        for t in range(n_rounds):
            cj = sid + t * NS

            @pl.when(cj < n_zchunks)
            def _():
                pltpu.sync_copy(rows0.at[pl.ds(0, zk)],
                                shw.at[pl.ds(cj * zk, zk)])
        plsc.subcore_barrier()

        def sidx(j):
            return src_v.at[pl.ds(j * k, k)]

        pltpu.async_copy(ml_hbm.at[sidx(0)], rows0, sem0)

        def chunk_body(t, carry):
            j0 = t * 2
            pltpu.async_copy(ml_hbm.at[sidx(j0 + 1)], rows1, sem1)
            pltpu.make_async_copy(ml_hbm.at[sidx(j0)], rows0, sem0).wait()
            pltpu.sync_copy(rows0, shw.at[dst_v.at[j0]], add=True)

            @pl.when(j0 + 2 < n_chunks)
            def _():
                pltpu.async_copy(ml_hbm.at[sidx(j0 + 2)], rows0, sem0)
            pltpu.make_async_copy(ml_hbm.at[sidx(j0 + 1)], rows1, sem1).wait()
            pltpu.sync_copy(rows1, shw.at[dst_v.at[j0 + 1]], add=True)
            return carry
        lax.fori_loop(0, n_chunks // 2, chunk_body, 0)
        if n_chunks % 2:
            jl = n_chunks - 1
            pltpu.make_async_copy(ml_hbm.at[sidx(jl)], rows0, sem0).wait()
            pltpu.sync_copy(rows0, shw.at[dst_v.at[jl]], add=True)
        plsc.subcore_barrier()

        for t in range(n_rounds + 2):
            if t >= 2:
                tw = t - 2
                cjw = sid + tw * NS
                bufw = rows0 if tw % 2 == 0 else rows1
                semw = sem0 if tw % 2 == 0 else sem1

                @pl.when(cjw < n_zchunks)
                def _():
                    pltpu.make_async_copy(
                        bufw.at[pl.ds(0, zk)],
                        out_hbm.at[cid, pl.ds(cjw * zk, zk)], semw).wait()
            if t < n_rounds:
                cj = sid + t * NS
                buf = rows0 if t % 2 == 0 else rows1
                sem = sem0 if t % 2 == 0 else sem1

                @pl.when(cj < n_zchunks)
                def _():
                    pltpu.sync_copy(shw.at[pl.ds(cj * zk, zk)],
                                    buf.at[pl.ds(0, zk)])
                    pltpu.async_copy(buf.at[pl.ds(0, zk)],
                                     out_hbm.at[cid, pl.ds(cj * zk, zk)], sem)

    return sc_w


def kernel(x, p, edge_index, Wg1, bg1, Wg2, bg2, Wl1, bl1, Wl2, bl2):
    n, d = x.shape
    e = edge_index.shape[1]
    dg_o = Wg2.shape[1]
    dl_h = Wl1.shape[1]
    dl_o = Wl2.shape[1]

    src = edge_index[0]
    dst = edge_index[1]
    evals = p[:, 0]
    Wl1a = Wl1[:d]
    Wl1b = Wl1[d:]

    e_per_w = e // NW
    k = 80
    n_chunks = e_per_w // k
    src_w = src.reshape(NW, e_per_w)
    dst_w = dst.reshape(NW, e_per_w)
    dst_c = dst.reshape(NW, n_chunks, k)

    bn = 2000

    m_g, h1l = pl.pallas_call(
        _d1_body,
        grid=(n // bn,),
        in_specs=[
            pl.BlockSpec((bn, d), lambda i: (i, 0)),
            pl.BlockSpec(Wg1.shape, lambda i: (0, 0)),
            pl.BlockSpec((1, Wg1.shape[1]), lambda i: (0, 0)),
            pl.BlockSpec(Wg2.shape, lambda i: (0, 0)),
            pl.BlockSpec((1, dg_o), lambda i: (0, 0)),
            pl.BlockSpec((d, dl_h), lambda i: (0, 0)),
        ],
        out_specs=[
            pl.BlockSpec((bn, dg_o), lambda i: (i, 0)),
            pl.BlockSpec((bn, dl_h), lambda i: (i, 0)),
        ],
        out_shape=[
            jax.ShapeDtypeStruct((n, dg_o), jnp.float32),
            jax.ShapeDtypeStruct((n, dl_h), jnp.float32),
        ],
    )(x, Wg1, bg1.reshape(1, -1), Wg2, bg2.reshape(1, -1), Wl1a)

    c_part = _make_sc_c(n, e_per_w)(src_w, dst_w, evals)

    bias2 = pl.pallas_call(
        _d2a_body,
        in_specs=[
            pl.BlockSpec((NW, n), lambda: (0, 0)),
            pl.BlockSpec((n, dg_o), lambda: (0, 0)),
            pl.BlockSpec((dg_o, dl_h), lambda: (0, 0)),
            pl.BlockSpec((1, dl_h), lambda: (0, 0)),
        ],
        out_specs=pl.BlockSpec((1, dl_h), lambda: (0, 0)),
        out_shape=jax.ShapeDtypeStruct((1, dl_h), jnp.float32),
    )(c_part, m_g, Wl1b, bl1.reshape(1, -1))

    m_l = pl.pallas_call(
        _d2b_body,
        grid=(n // bn,),
        in_specs=[
            pl.BlockSpec((bn, dl_h), lambda i: (i, 0)),
            pl.BlockSpec((1, dl_h), lambda i: (0, 0)),
            pl.BlockSpec((dl_h, dl_o), lambda i: (0, 0)),
            pl.BlockSpec((1, dl_o), lambda i: (0, 0)),
        ],
        out_specs=pl.BlockSpec((bn, dl_o), lambda i: (i, 0)),
        out_shape=jax.ShapeDtypeStruct((n, dl_o), jnp.float32),
    )(h1l, bias2, Wl2, bl2.reshape(1, -1))

    w_part = _make_sc_w(n, dl_o, n_chunks, k)(src_w, dst_c, m_l)

    w = pl.pallas_call(
        _sum_body,
        grid=(n // bn,),
        in_specs=[
            pl.BlockSpec((1, bn, dl_o), lambda i: (0, i, 0)),
            pl.BlockSpec((1, bn, dl_o), lambda i: (1, i, 0)),
        ],
        out_specs=pl.BlockSpec((bn, dl_o), lambda i: (i, 0)),
        out_shape=jax.ShapeDtypeStruct((n, dl_o), jnp.float32),
    )(w_part, w_part)

    return w


# speedup vs baseline: 17.8839x; 1.0105x over previous
import functools

import jax
import jax.numpy as jnp
from jax import lax
from jax.experimental import pallas as pl
from jax.experimental.pallas import tpu as pltpu
from jax.experimental.pallas import tpu_sc as plsc

NC = 2
NS = 16
NW = NC * NS
L = 16


def _d1_body(x_ref, wg1_ref, bg1_ref, wg2_ref, bg2_ref, wl1a_ref,
             mg_ref, h1l_ref):
    xb = x_ref[...]
    h = jnp.maximum(
        jnp.dot(xb, wg1_ref[...], preferred_element_type=jnp.float32)
        + bg1_ref[...], 0.0)
    mg_ref[...] = (jnp.dot(h, wg2_ref[...], preferred_element_type=jnp.float32)
                   + bg2_ref[...])
    h1l_ref[...] = jnp.dot(xb, wl1a_ref[...],
                           preferred_element_type=jnp.float32)


def _d2_body(cpart_ref, mg_ref, wl1b_ref, bl1_ref, h1l_ref, wl2_ref, bl2_ref,
             out_ref, bias_s):
    @pl.when(pl.program_id(0) == 0)
    def _():
        c = jnp.sum(cpart_ref[...], axis=0, keepdims=True)
        s = jnp.dot(c, mg_ref[...],
                    preferred_element_type=jnp.float32)
        bias_s[...] = (jnp.dot(s, wl1b_ref[...],
                               preferred_element_type=jnp.float32)
                       + bl1_ref[...])

    h = jnp.maximum(h1l_ref[...] + bias_s[...], 0.0)
    out_ref[...] = (jnp.dot(h, wl2_ref[...],
                            preferred_element_type=jnp.float32)
                    + bl2_ref[...])


def _sum_body(a_ref, b_ref, o_ref):
    o_ref[...] = a_ref[0] + b_ref[0]


def _make_sc_c(n_nodes, e_per_w):
    mesh = plsc.VectorSubcoreMesh(core_axis_name="c", subcore_axis_name="s",
                                  num_cores=NC, num_subcores=NS)

    @functools.partial(
        pl.kernel,
        out_type=jax.ShapeDtypeStruct((NW, n_nodes), jnp.float32),
        mesh=mesh,
        scratch_types=[
            pltpu.VMEM((n_nodes,), jnp.float32),
            pltpu.VMEM((e_per_w,), jnp.int32),
            pltpu.VMEM((e_per_w,), jnp.int32),
            pltpu.VMEM((n_nodes,), jnp.float32),
        ],
        compiler_params=pltpu.CompilerParams(needs_layout_passes=False),
    )
    def sc_c(src_hbm, dst_hbm, evals_hbm, out_hbm, evals_v, src_v, dst_v, c_v):
        cid = lax.axis_index("c")
        sid = lax.axis_index("s")
        wid = sid * NC + cid
        pltpu.sync_copy(evals_hbm, evals_v)
        pltpu.sync_copy(src_hbm.at[wid], src_v)
        pltpu.sync_copy(dst_hbm.at[wid], dst_v)

        def zero_body(i, carry):
            c_v[pl.ds(i * L, L)] = jnp.zeros((L,), jnp.float32)
            return carry
        lax.fori_loop(0, n_nodes // L, zero_body, 0)

        def body(i, carry):
            sv = src_v[pl.ds(i * L, L)]
            dv = dst_v[pl.ds(i * L, L)]
            vals = plsc.load_gather(evals_v, [dv])
            plsc.addupdate_scatter(c_v, [sv], vals)
            return carry
        lax.fori_loop(0, e_per_w // L, body, 0)
        pltpu.sync_copy(c_v, out_hbm.at[wid])

    return sc_c


def _make_sc_w(n_nodes, d_out, n_chunks, k):
    mesh = plsc.VectorSubcoreMesh(core_axis_name="c", subcore_axis_name="s",
                                  num_cores=NC, num_subcores=NS)
    zk = 80
    n_zchunks = n_nodes // zk

    @functools.partial(
        pl.kernel,
        out_type=jax.ShapeDtypeStruct((NC, n_nodes, d_out), jnp.float32),
        mesh=mesh,
        scratch_types=[
            pltpu.VMEM((n_chunks * k,), jnp.int32),
            pltpu.VMEM((n_chunks, k), jnp.int32),
            pltpu.VMEM((k, d_out), jnp.float32),
            pltpu.VMEM((k, d_out), jnp.float32),
            pltpu.VMEM_SHARED((n_nodes, d_out), jnp.float32),
            pltpu.SemaphoreType.DMA,
            pltpu.SemaphoreType.DMA,
        ],
        compiler_params=pltpu.CompilerParams(needs_layout_passes=False),
    )
    def sc_w(src_hbm, dst_hbm, ml_hbm, out_hbm, src_v, dst_v, rows0, rows1,
             shw, sem0, sem1):
        cid = lax.axis_index("c")
        sid = lax.axis_index("s")
        wid = sid * NC + cid
        pltpu.sync_copy(src_hbm.at[wid], src_v)
        pltpu.sync_copy(dst_hbm.at[wid], dst_v)

        def zero_body(i, carry):
            rows0[i // (d_out // L), pl.ds((i % (d_out // L)) * L, L)] = (
                jnp.zeros((L,), jnp.float32))
            return carry
        lax.fori_loop(0, k * (d_out // L), zero_body, 0)
        n_rounds = (n_zchunks + NS - 1) // NS
        for t in range(n_rounds):
            cj = sid + t * NS

            @pl.when(cj < n_zchunks)
            def _():
                pltpu.sync_copy(rows0.at[pl.ds(0, zk)],
                                shw.at[pl.ds(cj * zk, zk)])
        plsc.subcore_barrier()

        def sidx(j):
            return src_v.at[pl.ds(j * k, k)]

        pltpu.async_copy(ml_hbm.at[sidx(0)], rows0, sem0)

        def chunk_body(t, carry):
            j0 = t * 2
            pltpu.async_copy(ml_hbm.at[sidx(j0 + 1)], rows1, sem1)
            pltpu.make_async_copy(ml_hbm.at[sidx(j0)], rows0, sem0).wait()
            pltpu.sync_copy(rows0, shw.at[dst_v.at[j0]], add=True)

            @pl.when(j0 + 2 < n_chunks)
            def _():
                pltpu.async_copy(ml_hbm.at[sidx(j0 + 2)], rows0, sem0)
            pltpu.make_async_copy(ml_hbm.at[sidx(j0 + 1)], rows1, sem1).wait()
            pltpu.sync_copy(rows1, shw.at[dst_v.at[j0 + 1]], add=True)
            return carry
        lax.fori_loop(0, n_chunks // 2, chunk_body, 0)
        if n_chunks % 2:
            jl = n_chunks - 1
            pltpu.make_async_copy(ml_hbm.at[sidx(jl)], rows0, sem0).wait()
            pltpu.sync_copy(rows0, shw.at[dst_v.at[jl]], add=True)
        plsc.subcore_barrier()

        for t in range(n_rounds + 2):
            if t >= 2:
                tw = t - 2
                cjw = sid + tw * NS
                bufw = rows0 if tw % 2 == 0 else rows1
                semw = sem0 if tw % 2 == 0 else sem1

                @pl.when(cjw < n_zchunks)
                def _():
                    pltpu.make_async_copy(
                        bufw.at[pl.ds(0, zk)],
                        out_hbm.at[cid, pl.ds(cjw * zk, zk)], semw).wait()
            if t < n_rounds:
                cj = sid + t * NS
                buf = rows0 if t % 2 == 0 else rows1
                sem = sem0 if t % 2 == 0 else sem1

                @pl.when(cj < n_zchunks)
                def _():
                    pltpu.sync_copy(shw.at[pl.ds(cj * zk, zk)],
                                    buf.at[pl.ds(0, zk)])
                    pltpu.async_copy(buf.at[pl.ds(0, zk)],
                                     out_hbm.at[cid, pl.ds(cj * zk, zk)], sem)

    return sc_w


def kernel(x, p, edge_index, Wg1, bg1, Wg2, bg2, Wl1, bl1, Wl2, bl2):
    n, d = x.shape
    e = edge_index.shape[1]
    dg_o = Wg2.shape[1]
    dl_h = Wl1.shape[1]
    dl_o = Wl2.shape[1]

    src = edge_index[0]
    dst = edge_index[1]
    evals = p[:, 0]
    Wl1a = Wl1[:d]
    Wl1b = Wl1[d:]

    e_per_w = e // NW
    k = 80
    n_chunks = e_per_w // k
    src_w = src.reshape(NW, e_per_w)
    dst_w = dst.reshape(NW, e_per_w)
    dst_c = dst.reshape(NW, n_chunks, k)

    bn = 2000

    m_g, h1l = pl.pallas_call(
        _d1_body,
        grid=(n // bn,),
        in_specs=[
            pl.BlockSpec((bn, d), lambda i: (i, 0)),
            pl.BlockSpec(Wg1.shape, lambda i: (0, 0)),
            pl.BlockSpec((1, Wg1.shape[1]), lambda i: (0, 0)),
            pl.BlockSpec(Wg2.shape, lambda i: (0, 0)),
            pl.BlockSpec((1, dg_o), lambda i: (0, 0)),
            pl.BlockSpec((d, dl_h), lambda i: (0, 0)),
        ],
        out_specs=[
            pl.BlockSpec((bn, dg_o), lambda i: (i, 0)),
            pl.BlockSpec((bn, dl_h), lambda i: (i, 0)),
        ],
        out_shape=[
            jax.ShapeDtypeStruct((n, dg_o), jnp.float32),
            jax.ShapeDtypeStruct((n, dl_h), jnp.float32),
        ],
    )(x, Wg1, bg1.reshape(1, -1), Wg2, bg2.reshape(1, -1), Wl1a)

    c_part = _make_sc_c(n, e_per_w)(src_w, dst_w, evals)

    m_l = pl.pallas_call(
        _d2_body,
        grid=(n // bn,),
        in_specs=[
            pl.BlockSpec((NW, n), lambda i: (0, 0)),
            pl.BlockSpec((n, dg_o), lambda i: (0, 0)),
            pl.BlockSpec((dg_o, dl_h), lambda i: (0, 0)),
            pl.BlockSpec((1, dl_h), lambda i: (0, 0)),
            pl.BlockSpec((bn, dl_h), lambda i: (i, 0)),
            pl.BlockSpec((dl_h, dl_o), lambda i: (0, 0)),
            pl.BlockSpec((1, dl_o), lambda i: (0, 0)),
        ],
        out_specs=pl.BlockSpec((bn, dl_o), lambda i: (i, 0)),
        out_shape=jax.ShapeDtypeStruct((n, dl_o), jnp.float32),
        scratch_shapes=[pltpu.VMEM((1, dl_h), jnp.float32)],
    )(c_part, m_g, Wl1b, bl1.reshape(1, -1), h1l, Wl2, bl2.reshape(1, -1))

    w_part = _make_sc_w(n, dl_o, n_chunks, k)(src_w, dst_c, m_l)

    w = pl.pallas_call(
        _sum_body,
        grid=(n // bn,),
        in_specs=[
            pl.BlockSpec((1, bn, dl_o), lambda i: (0, i, 0)),
            pl.BlockSpec((1, bn, dl_o), lambda i: (1, i, 0)),
        ],
        out_specs=pl.BlockSpec((bn, dl_o), lambda i: (i, 0)),
        out_shape=jax.ShapeDtypeStruct((n, dl_o), jnp.float32),
    )(w_part, w_part)

    return w
